# baseline (speedup 1.0000x reference)
import functools

import jax
import jax.numpy as jnp
from jax import lax
from jax.experimental import pallas as pl
from jax.experimental.pallas import tpu as pltpu
from jax.experimental.pallas import tpu_sc as plsc

N = 100000
D = 32
NC = 2
NS = 16
NW = NC * NS
CHUNK = 128
K = 25
B_PER_W = K * CHUNK
B_PAD = NW * B_PER_W

_mesh = plsc.VectorSubcoreMesh(core_axis_name="c", subcore_axis_name="s")


@functools.partial(
    pl.kernel,
    mesh=_mesh,
    out_type=jax.ShapeDtypeStruct((NW, K, CHUNK, D), jnp.float32),
    compiler_params=pltpu.CompilerParams(use_tc_tiling_on_sc=False),
    scratch_types=[
        pltpu.VMEM((K, CHUNK), jnp.int32),
        pltpu.VMEM((K, CHUNK, D), jnp.float32),
        pltpu.SemaphoreType.DMA,
    ],
)
def _gather_kernel(idx_hbm, table_hbm, out_hbm, idx_v, rows_v, sem):
    wid = lax.axis_index("s") * NC + lax.axis_index("c")
    pltpu.sync_copy(idx_hbm.at[wid], idx_v)

    def fire(j, carry):
        pltpu.make_async_copy(table_hbm.at[idx_v.at[j]], rows_v.at[j], sem).start()
        return carry

    lax.fori_loop(0, K, fire, 0)

    def drain(j, carry):
        pltpu.make_async_copy(table_hbm.at[idx_v.at[j]], rows_v.at[j], sem).wait()
        return carry

    lax.fori_loop(0, K, drain, 0)

    pltpu.sync_copy(rows_v, out_hbm.at[wid])


def kernel(node_feature, table):
    idx = node_feature[:, 0].astype(jnp.int32)
    idx = jnp.concatenate([idx, jnp.zeros((B_PAD - N,), jnp.int32)])
    out = _gather_kernel(idx.reshape(NW, K, CHUNK), table)
    return out.reshape(B_PAD, D)[:N]
